# baseline (speedup 1.0000x reference)
import functools

import jax
import jax.numpy as jnp
from jax import lax
from jax.experimental import pallas as pl
from jax.experimental.pallas import tpu as pltpu
from jax.experimental.pallas import tpu_sc as plsc

B = 16384
D = 32
NC = 2
NS = 16
NW = NC * NS
BPW = B // NW
SEG = 128
NSEG = BPW // SEG

ROWS_TC = 2048


def _gather_body(uidx_hbm, iidx_hbm, embu_hbm, embi_hbm, gu_hbm, gi_hbm,
                 uidx_v, iidx_v, urows_v, irows_v, usem, isem):
    c = lax.axis_index("c")
    s = lax.axis_index("s")
    wid = s * NC + c

    pltpu.sync_copy(uidx_hbm.at[wid], uidx_v)
    pltpu.sync_copy(iidx_hbm.at[wid], iidx_v)

    copies = []
    for g in range(NSEG):
        copies.append(pltpu.async_copy(
            embu_hbm.at[uidx_v.at[g]], urows_v.at[pl.ds(g * SEG, SEG)], usem))
        copies.append(pltpu.async_copy(
            embi_hbm.at[iidx_v.at[g]], irows_v.at[pl.ds(g * SEG, SEG)], isem))
    for cp in copies:
        cp.wait()

    base = wid * BPW
    pltpu.sync_copy(urows_v, gu_hbm.at[pl.ds(base, BPW)])
    pltpu.sync_copy(irows_v, gi_hbm.at[pl.ds(base, BPW)])


def _mlp_body(gu_ref, gi_ref, w1ut_ref, w1it_ref, b1_ref, w2t_ref, b2_ref,
              wat_ref, ba_ref, out_ref):
    gu = gu_ref[...]
    gi = gi_ref[...]
    h1 = (jnp.dot(gu, w1ut_ref[...], preferred_element_type=jnp.float32)
          + jnp.dot(gi, w1it_ref[...], preferred_element_type=jnp.float32)
          + b1_ref[...])
    h1 = jnp.maximum(h1, 0.0)
    h2 = jnp.dot(h1, w2t_ref[...], preferred_element_type=jnp.float32) + b2_ref[...]
    h2 = jnp.maximum(h2, 0.0)
    logits = jnp.dot(h2, wat_ref[...], preferred_element_type=jnp.float32) + ba_ref[0, 0]
    out_ref[...] = 1.0 / (1.0 + jnp.exp(-logits))


def kernel(user_indices, item_indices, emb_user, emb_item, W1, b1, W2, b2, Wa, ba):
    uidx = user_indices.reshape(NW, NSEG, SEG)
    iidx = item_indices.reshape(NW, NSEG, SEG)

    gather = pl.kernel(
        _gather_body,
        out_type=(jax.ShapeDtypeStruct((B, D), jnp.float32),
                  jax.ShapeDtypeStruct((B, D), jnp.float32)),
        mesh=plsc.VectorSubcoreMesh(core_axis_name="c", subcore_axis_name="s"),
        compiler_params=pltpu.CompilerParams(use_tc_tiling_on_sc=False),
        scratch_types=[
            pltpu.VMEM((NSEG, SEG), jnp.int32),
            pltpu.VMEM((NSEG, SEG), jnp.int32),
            pltpu.VMEM((BPW, D), jnp.float32),
            pltpu.VMEM((BPW, D), jnp.float32),
            pltpu.SemaphoreType.DMA,
            pltpu.SemaphoreType.DMA,
        ],
    )
    gu, gi = gather(uidx, iidx, emb_user, emb_item)

    grid = (B // ROWS_TC,)
    full = lambda s: pl.BlockSpec(s, lambda i: (0, 0))
    out = pl.pallas_call(
        _mlp_body,
        grid=grid,
        in_specs=[
            pl.BlockSpec((ROWS_TC, D), lambda i: (i, 0)),
            pl.BlockSpec((ROWS_TC, D), lambda i: (i, 0)),
            full((D, 8)),
            full((D, 8)),
            full((1, 8)),
            full((8, 8)),
            full((1, 8)),
            full((8, 1)),
            full((1, 1)),
        ],
        out_specs=pl.BlockSpec((ROWS_TC, 1), lambda i: (i, 0)),
        out_shape=jax.ShapeDtypeStruct((B, 1), jnp.float32),
    )(gu, gi,
      W1[:, :D].T, W1[:, D:].T, b1.reshape(1, 8),
      W2.T, b2.reshape(1, 8), Wa.T, ba.reshape(1, 1))
    return out
